# baseline (speedup 1.0000x reference)
import math

import jax
import jax.numpy as jnp
from jax.experimental import pallas as pl
from jax.experimental.pallas import tpu as pltpu

_N = 8192
_D = 16
_BM = 512
_BM1 = 256
_NT = 3


def _step1_kern(coef_ref, a_ref, p_ref, u_ref, s_ref,
                u16_ref, s16_ref, ap16_ref):
    a16 = a_ref[...].astype(jnp.bfloat16)
    p16 = p_ref[...].astype(jnp.bfloat16)
    ap16_ref[:, 0:_N] = a16
    ap16_ref[:, _N:2 * _N] = p16
    au = jnp.dot(a16, u_ref[...], preferred_element_type=jnp.float32)
    ps = jnp.dot(p16, s_ref[...], preferred_element_type=jnp.float32)
    u16_ref[...] = au.astype(jnp.bfloat16)
    s16_ref[...] = (coef_ref[0] * au + ps).astype(jnp.bfloat16)


def _steps_kern(coef_ref, ap_ref, u0_ref, s0_ref, y_ref, u_scr, s_scr):
    t = pl.program_id(0)
    i = pl.program_id(1)

    @pl.when(jnp.logical_and(t == 0, i == 0))
    def _():
        u_scr[1] = u0_ref[...]
        s_scr[1] = s0_ref[...]

    rslot = jax.lax.rem(t + 1, 2)
    wslot = jax.lax.rem(t, 2)
    au = jnp.dot(ap_ref[:, 0:_N], u_scr[rslot],
                 preferred_element_type=jnp.float32)
    ps = jnp.dot(ap_ref[:, _N:2 * _N], s_scr[rslot],
                 preferred_element_type=jnp.float32)
    s_new = coef_ref[t] * au + ps
    obase = i * _BM
    u_scr[wslot, pl.ds(obase, _BM), :] = au.astype(jnp.bfloat16)
    s_scr[wslot, pl.ds(obase, _BM), :] = s_new.astype(jnp.bfloat16)

    @pl.when(t == _NT - 1)
    def _():
        y_ref[...] = s_new


_step1 = pl.pallas_call(
    _step1_kern,
    grid=(_N // _BM1,),
    in_specs=[
        pl.BlockSpec(memory_space=pltpu.SMEM),
        pl.BlockSpec((_BM1, _N), lambda i: (i, 0)),
        pl.BlockSpec((_BM1, _N), lambda i: (i, 0)),
        pl.BlockSpec((_N, _D), lambda i: (0, 0)),
        pl.BlockSpec((_N, _D), lambda i: (0, 0)),
    ],
    out_specs=[
        pl.BlockSpec((_BM1, _D), lambda i: (i, 0)),
        pl.BlockSpec((_BM1, _D), lambda i: (i, 0)),
        pl.BlockSpec((_BM1, 2 * _N), lambda i: (i, 0)),
    ],
    out_shape=[
        jax.ShapeDtypeStruct((_N, _D), jnp.bfloat16),
        jax.ShapeDtypeStruct((_N, _D), jnp.bfloat16),
        jax.ShapeDtypeStruct((_N, 2 * _N), jnp.bfloat16),
    ],
    compiler_params=pltpu.CompilerParams(
        dimension_semantics=("arbitrary",),
    ),
)

_steps234 = pl.pallas_call(
    _steps_kern,
    grid=(_NT, _N // _BM),
    in_specs=[
        pl.BlockSpec(memory_space=pltpu.SMEM),
        pl.BlockSpec((_BM, 2 * _N), lambda t, i: (i, 0)),
        pl.BlockSpec((_N, _D), lambda t, i: (0, 0)),
        pl.BlockSpec((_N, _D), lambda t, i: (0, 0)),
    ],
    out_specs=pl.BlockSpec((_BM, _D), lambda t, i: (i, 0)),
    out_shape=jax.ShapeDtypeStruct((_N, _D), jnp.float32),
    scratch_shapes=[
        pltpu.VMEM((2, _N, _D), jnp.bfloat16),
        pltpu.VMEM((2, _N, _D), jnp.bfloat16),
    ],
    compiler_params=pltpu.CompilerParams(
        dimension_semantics=("arbitrary", "arbitrary"),
    ),
)


def kernel(x, adj, poly_item, filter_param):
    k = filter_param.shape[0] - 1
    fp = jax.nn.relu(filter_param)[:, 0]
    coefs = [math.comb(k, i) / (2.0 ** k) for i in range(k + 1)]
    c3 = (coefs[k - 1] * fp[k - 1]).reshape(1)
    u16, s16, ap16 = _step1(c3, adj, poly_item,
                            x.astype(jnp.bfloat16),
                            (coefs[k] * fp[k] * x).astype(jnp.bfloat16))
    cvec = jnp.stack([coefs[2] * fp[2], coefs[1] * fp[1], coefs[0] * fp[0]])
    y = _steps234(cvec, ap16, u16, s16)
    return y
